# baseline (speedup 1.0000x reference)
import functools

import jax
import jax.numpy as jnp
from jax import lax
from jax.experimental import pallas as pl
from jax.experimental.pallas import tpu as pltpu
from jax.experimental.pallas import tpu_sc as plsc

N = 10000
C = 128
HEADS = 6
E = 320000
TRASH = N
NPAD = 10240
ROWS_PER_TILE = NPAD // 16

DCH = 128
DCPT = 79

ACH = 80

_mesh = plsc.VectorSubcoreMesh(core_axis_name="c", subcore_axis_name="s")


def _fill(ref, nvec, value, dtype):
    v = jnp.full((16,), value, dtype)
    for j in range(nvec):
        ref[pl.ds(j * 16, 16)] = v


def _masked_targets(rv, cv, av, nvec):
    for j in range(nvec):
        r = rv[pl.ds(j * 16, 16)]
        c = cv[pl.ds(j * 16, 16)]
        d = r - c
        t = jnp.right_shift(d | (-d), 31)
        av[pl.ds(j * 16, 16)] = (c & t) | (TRASH & ~t)


def _deg_body(row_hbm, col_hbm, deg_out,
              rbig, cbig, av0, av1, ones_v, zrow_v, acc, ss0, ss1):
    cid = lax.axis_index("c")
    sid = lax.axis_index("s")
    wid = cid * 16 + sid
    is31 = wid == 31
    n = jnp.where(is31, 51, DCPT)
    nq = jnp.where(is31, 24, 38)
    in_base = wid * (DCPT * DCH)
    load_base = jnp.minimum(in_base, E - DCPT * DCH)
    shift = in_base - load_base

    _fill(ones_v, DCH // 16, 1.0, jnp.float32)
    _fill(zrow_v, ROWS_PER_TILE // 16, 0.0, jnp.float32)
    pltpu.sync_copy(row_hbm.at[pl.ds(load_base, DCPT * DCH)], rbig)
    pltpu.sync_copy(col_hbm.at[pl.ds(load_base, DCPT * DCH)], cbig)
    pltpu.sync_copy(zrow_v, acc.at[pl.ds(sid * ROWS_PER_TILE, ROWS_PER_TILE)])
    plsc.subcore_barrier()

    av = (av0, av1)
    ss = (ss0, ss1)

    def do_chunk(i, b, first):
        if not first:
            pltpu.make_async_copy(ones_v, acc.at[av[b]], ss[b]).wait()
        for j in range(DCH // 16):
            o = shift + i * DCH + j * 16
            r = rbig[pl.ds(o, 16)]
            c = cbig[pl.ds(o, 16)]
            d = r - c
            t = jnp.right_shift(d | (-d), 31)
            av[b][pl.ds(j * 16, 16)] = (c & t) | (TRASH & ~t)
        pltpu.async_copy(ones_v, acc.at[av[b]], ss[b], add=True)

    do_chunk(0, 0, True)
    do_chunk(1, 1, True)

    def pair_body(m, carry):
        for b in range(2):
            do_chunk(m * 2 + b, b, False)
        return carry

    lax.fori_loop(1, 1 + nq, pair_body, 0)
    do_chunk(n - 1, 0, False)
    pltpu.make_async_copy(ones_v, acc.at[av[1]], ss1).wait()
    pltpu.make_async_copy(ones_v, acc.at[av[0]], ss0).wait()
    plsc.subcore_barrier()
    pltpu.sync_copy(acc.at[pl.ds(sid * ROWS_PER_TILE, ROWS_PER_TILE)],
                    deg_out.at[cid, pl.ds(sid * ROWS_PER_TILE, ROWS_PER_TILE)])


_deg_call = functools.partial(
    pl.kernel,
    out_type=jax.ShapeDtypeStruct((2, NPAD), jnp.float32),
    mesh=_mesh,
    scratch_types=[
        pltpu.VMEM((DCPT * DCH,), jnp.int32),
        pltpu.VMEM((DCPT * DCH,), jnp.int32),
        pltpu.VMEM((DCH,), jnp.int32),
        pltpu.VMEM((DCH,), jnp.int32),
        pltpu.VMEM((DCH,), jnp.float32),
        pltpu.VMEM((ROWS_PER_TILE,), jnp.float32),
        pltpu.VMEM_SHARED((NPAD,), jnp.float32),
        pltpu.SemaphoreType.DMA,
        pltpu.SemaphoreType.DMA,
    ],
)(_deg_body)


def _agg_body(g_hbm, row_hbm, col_hbm, out_hbm,
              rv0, rv1, rv2, rv3, cv0, cv1, cv2, cv3, av0, av1, av2, av3,
              b0, b1, b2, b3, acc,
              sg0, sg1, sg2, sg3, ss0, ss1, ss2, ss3):
    cid = lax.axis_index("c")
    sid = lax.axis_index("s")
    n = jnp.where(cid == 0, 156, jnp.where(sid < 8, 96, 92))
    base_chunk = jnp.where(cid == 0,
                           sid * 156,
                           2496 + sid * 92 + 4 * jnp.minimum(sid, 8))
    base = base_chunk * ACH

    rv = (rv0, rv1, rv2, rv3)
    cv = (cv0, cv1, cv2, cv3)
    av = (av0, av1, av2, av3)
    bufs = (b0, b1, b2, b3)
    sg = (sg0, sg1, sg2, sg3)
    ss = (ss0, ss1, ss2, ss3)

    zeros16 = jnp.zeros((16,), jnp.float32)

    def zrow(i, carry):
        for j in range(C // 16):
            b0[i, pl.ds(j * 16, 16)] = zeros16
        return carry

    lax.fori_loop(0, ACH, zrow, 0)

    def zacc(k, carry):
        pltpu.sync_copy(
            b0, acc.at[pl.ds(sid * ROWS_PER_TILE + k * ACH, ACH)])
        return carry

    lax.fori_loop(0, ROWS_PER_TILE // ACH, zacc, 0)
    plsc.subcore_barrier()

    def load_and_gather(j, q):
        pltpu.sync_copy(row_hbm.at[pl.ds(base + j * ACH, ACH)], rv[q])
        pltpu.sync_copy(col_hbm.at[pl.ds(base + j * ACH, ACH)], cv[q])
        _masked_targets(rv[q], cv[q], av[q], ACH // 16)
        pltpu.async_copy(g_hbm.at[rv[q]], bufs[q], sg[q])

    def finish_chunk(i, q):
        pltpu.make_async_copy(g_hbm.at[rv[q]], bufs[q], sg[q]).wait()
        pltpu.async_copy(bufs[q], acc.at[av[q]], ss[q], add=True)

    load_and_gather(0, 0)
    load_and_gather(1, 1)
    for q in range(4):
        i = q
        q2 = (q + 2) % 4
        if q >= 2:
            pltpu.make_async_copy(bufs[q2], acc.at[av[q2]], ss[q2]).wait()
        load_and_gather(i + 2, q2)
        finish_chunk(i, q)

    def quad_body(m, carry):
        for q in range(4):
            i = m * 4 + q
            q2 = (q + 2) % 4
            pltpu.make_async_copy(bufs[q2], acc.at[av[q2]], ss[q2]).wait()
            load_and_gather(i + 2, q2)
            finish_chunk(i, q)
        return carry

    lax.fori_loop(1, n // 4 - 1, quad_body, 0)
    for q in range(4):
        i = n - 4 + q
        q2 = (q + 2) % 4
        pltpu.make_async_copy(bufs[q2], acc.at[av[q2]], ss[q2]).wait()
        if q < 2:
            load_and_gather(i + 2, q2)
        finish_chunk(i, q)
    for q in (2, 3):
        pltpu.make_async_copy(bufs[q], acc.at[av[q]], ss[q]).wait()

    plsc.subcore_barrier()
    pltpu.sync_copy(acc.at[pl.ds(sid * ROWS_PER_TILE, ROWS_PER_TILE)],
                    out_hbm.at[cid, pl.ds(sid * ROWS_PER_TILE, ROWS_PER_TILE)])


_agg_call = functools.partial(
    pl.kernel,
    out_type=jax.ShapeDtypeStruct((2, NPAD, C), jnp.float32),
    mesh=_mesh,
    scratch_types=(
        [pltpu.VMEM((ACH,), jnp.int32)] * 12
        + [pltpu.VMEM((ACH, C), jnp.float32)] * 4
        + [pltpu.VMEM_SHARED((NPAD, C), jnp.float32)]
        + [pltpu.SemaphoreType.DMA] * 8
    ),
)(_agg_body)


_RB = 640


def _g_body(deg_ref, x_ref, w_ref, g_ref):
    p = deg_ref[...]
    s = lax.rsqrt(p[0] + p[1] + 1.0)
    h = jnp.dot(x_ref[...], w_ref[...], preferred_element_type=jnp.float32)
    g_ref[...] = s[:, None] * h


def _g_call(deg_parts, x, W_gcn):
    return pl.pallas_call(
        _g_body,
        grid=(NPAD // _RB,),
        in_specs=[
            pl.BlockSpec((2, _RB), lambda i: (0, i)),
            pl.BlockSpec((_RB, C), lambda i: (i, 0)),
            pl.BlockSpec((C, C), lambda i: (0, 0)),
        ],
        out_specs=pl.BlockSpec((_RB, C), lambda i: (i, 0)),
        out_shape=jax.ShapeDtypeStruct((NPAD, C), jnp.float32),
    )(deg_parts, x, W_gcn)


def _out_body(deg_ref, a_ref, g_ref, b_ref, w_ref, o_ref):
    p = deg_ref[...]
    a = a_ref[...]
    s = lax.rsqrt(p[0] + p[1] + 1.0)
    out1 = s[:, None] * (a[0] + a[1] + g_ref[...]) + b_ref[...]
    o = jnp.dot(out1, w_ref[...], preferred_element_type=jnp.float32)
    o_ref[...] = o.reshape(_RB, HEADS, C)


def _out_call(deg_parts, a_parts, g, b2d, weight):
    return pl.pallas_call(
        _out_body,
        grid=(NPAD // _RB,),
        in_specs=[
            pl.BlockSpec((2, _RB), lambda i: (0, i)),
            pl.BlockSpec((2, _RB, C), lambda i: (0, i, 0)),
            pl.BlockSpec((_RB, C), lambda i: (i, 0)),
            pl.BlockSpec((1, C), lambda i: (0, 0)),
            pl.BlockSpec((C, HEADS * C), lambda i: (0, 0)),
        ],
        out_specs=pl.BlockSpec((_RB, HEADS, C), lambda i: (i, 0, 0)),
        out_shape=jax.ShapeDtypeStruct((N, HEADS, C), jnp.float32),
    )(deg_parts, a_parts, g, b2d, weight)


@jax.jit
def kernel(x, edge_index, old_index, W_gcn, b_gcn, weight):
    row1d = edge_index[0]
    col1d = edge_index[1]
    deg_parts = _deg_call(row1d, col1d)
    g = _g_call(deg_parts, x, W_gcn)
    a_parts = _agg_call(g, row1d, col1d)
    return _out_call(deg_parts, a_parts, g, b_gcn.reshape(1, C), weight)


# speedup vs baseline: 1.0438x; 1.0438x over previous
import functools

import jax
import jax.numpy as jnp
from jax import lax
from jax.experimental import pallas as pl
from jax.experimental.pallas import tpu as pltpu
from jax.experimental.pallas import tpu_sc as plsc

N = 10000
C = 128
HEADS = 6
E = 320000
TRASH = N
NPAD = 10240
ROWS_PER_TILE = NPAD // 16

DCH = 128
DCPT = 79

ACH = 80

_mesh = plsc.VectorSubcoreMesh(core_axis_name="c", subcore_axis_name="s")


def _fill(ref, nvec, value, dtype):
    v = jnp.full((16,), value, dtype)
    for j in range(nvec):
        ref[pl.ds(j * 16, 16)] = v


def _masked_targets(rv, cv, av, nvec):
    for j in range(nvec):
        r = rv[pl.ds(j * 16, 16)]
        c = cv[pl.ds(j * 16, 16)]
        d = r - c
        t = jnp.right_shift(d | (-d), 31)
        av[pl.ds(j * 16, 16)] = (c & t) | (TRASH & ~t)


def _deg_body(row_hbm, col_hbm, deg_out,
              rbig, cbig, av0, av1, ones_v, zrow_v, acc, ss0, ss1):
    cid = lax.axis_index("c")
    sid = lax.axis_index("s")
    wid = cid * 16 + sid
    is31 = wid == 31
    n = jnp.where(is31, 51, DCPT)
    nq = jnp.where(is31, 24, 38)
    in_base = wid * (DCPT * DCH)
    load_base = jnp.minimum(in_base, E - DCPT * DCH)
    shift = in_base - load_base

    _fill(ones_v, DCH // 16, 1.0, jnp.float32)
    _fill(zrow_v, ROWS_PER_TILE // 16, 0.0, jnp.float32)
    pltpu.sync_copy(row_hbm.at[pl.ds(load_base, DCPT * DCH)], rbig)
    pltpu.sync_copy(col_hbm.at[pl.ds(load_base, DCPT * DCH)], cbig)
    pltpu.sync_copy(zrow_v, acc.at[pl.ds(sid * ROWS_PER_TILE, ROWS_PER_TILE)])
    plsc.subcore_barrier()

    av = (av0, av1)
    ss = (ss0, ss1)

    def do_chunk(i, b, first):
        if not first:
            pltpu.make_async_copy(ones_v, acc.at[av[b]], ss[b]).wait()
        for j in range(DCH // 16):
            o = shift + i * DCH + j * 16
            r = rbig[pl.ds(o, 16)]
            c = cbig[pl.ds(o, 16)]
            d = r - c
            t = jnp.right_shift(d | (-d), 31)
            av[b][pl.ds(j * 16, 16)] = (c & t) | (TRASH & ~t)
        pltpu.async_copy(ones_v, acc.at[av[b]], ss[b], add=True)

    do_chunk(0, 0, True)
    do_chunk(1, 1, True)

    def pair_body(m, carry):
        for b in range(2):
            do_chunk(m * 2 + b, b, False)
        return carry

    lax.fori_loop(1, 1 + nq, pair_body, 0)
    do_chunk(n - 1, 0, False)
    pltpu.make_async_copy(ones_v, acc.at[av[1]], ss1).wait()
    pltpu.make_async_copy(ones_v, acc.at[av[0]], ss0).wait()
    plsc.subcore_barrier()
    pltpu.sync_copy(acc.at[pl.ds(sid * ROWS_PER_TILE, ROWS_PER_TILE)],
                    deg_out.at[cid, pl.ds(sid * ROWS_PER_TILE, ROWS_PER_TILE)])


_deg_call = functools.partial(
    pl.kernel,
    out_type=jax.ShapeDtypeStruct((2, NPAD), jnp.float32),
    mesh=_mesh,
    scratch_types=[
        pltpu.VMEM((DCPT * DCH,), jnp.int32),
        pltpu.VMEM((DCPT * DCH,), jnp.int32),
        pltpu.VMEM((DCH,), jnp.int32),
        pltpu.VMEM((DCH,), jnp.int32),
        pltpu.VMEM((DCH,), jnp.float32),
        pltpu.VMEM((ROWS_PER_TILE,), jnp.float32),
        pltpu.VMEM_SHARED((NPAD,), jnp.float32),
        pltpu.SemaphoreType.DMA,
        pltpu.SemaphoreType.DMA,
    ],
)(_deg_body)


def _agg_body(g_hbm, row_hbm, col_hbm, out_hbm,
              rv0, rv1, rv2, rv3, cv0, cv1, cv2, cv3, av0, av1, av2, av3,
              b0, b1, b2, b3, acc,
              sg0, sg1, sg2, sg3, ss0, ss1, ss2, ss3):
    cid = lax.axis_index("c")
    sid = lax.axis_index("s")
    n = jnp.where(cid == 0, 144, jnp.where(sid < 8, 108, 104))
    base_chunk = jnp.where(cid == 0,
                           sid * 144,
                           2304 + sid * 104 + 4 * jnp.minimum(sid, 8))
    base = base_chunk * ACH

    rv = (rv0, rv1, rv2, rv3)
    cv = (cv0, cv1, cv2, cv3)
    av = (av0, av1, av2, av3)
    bufs = (b0, b1, b2, b3)
    sg = (sg0, sg1, sg2, sg3)
    ss = (ss0, ss1, ss2, ss3)

    zeros16 = jnp.zeros((16,), jnp.float32)

    def zrow(i, carry):
        for j in range(C // 16):
            b0[i, pl.ds(j * 16, 16)] = zeros16
        return carry

    lax.fori_loop(0, ACH, zrow, 0)

    def zacc(k, carry):
        pltpu.sync_copy(
            b0, acc.at[pl.ds(sid * ROWS_PER_TILE + k * ACH, ACH)])
        return carry

    lax.fori_loop(0, ROWS_PER_TILE // ACH, zacc, 0)
    plsc.subcore_barrier()

    def load_and_gather(j, q):
        pltpu.sync_copy(row_hbm.at[pl.ds(base + j * ACH, ACH)], rv[q])
        pltpu.sync_copy(col_hbm.at[pl.ds(base + j * ACH, ACH)], cv[q])
        _masked_targets(rv[q], cv[q], av[q], ACH // 16)
        pltpu.async_copy(g_hbm.at[rv[q]], bufs[q], sg[q])

    def finish_chunk(i, q):
        pltpu.make_async_copy(g_hbm.at[rv[q]], bufs[q], sg[q]).wait()
        pltpu.async_copy(bufs[q], acc.at[av[q]], ss[q], add=True)

    load_and_gather(0, 0)
    load_and_gather(1, 1)
    for q in range(4):
        i = q
        q2 = (q + 2) % 4
        if q >= 2:
            pltpu.make_async_copy(bufs[q2], acc.at[av[q2]], ss[q2]).wait()
        load_and_gather(i + 2, q2)
        finish_chunk(i, q)

    def quad_body(m, carry):
        for q in range(4):
            i = m * 4 + q
            q2 = (q + 2) % 4
            pltpu.make_async_copy(bufs[q2], acc.at[av[q2]], ss[q2]).wait()
            load_and_gather(i + 2, q2)
            finish_chunk(i, q)
        return carry

    lax.fori_loop(1, n // 4 - 1, quad_body, 0)
    for q in range(4):
        i = n - 4 + q
        q2 = (q + 2) % 4
        pltpu.make_async_copy(bufs[q2], acc.at[av[q2]], ss[q2]).wait()
        if q < 2:
            load_and_gather(i + 2, q2)
        finish_chunk(i, q)
    for q in (2, 3):
        pltpu.make_async_copy(bufs[q], acc.at[av[q]], ss[q]).wait()

    plsc.subcore_barrier()
    pltpu.sync_copy(acc.at[pl.ds(sid * ROWS_PER_TILE, ROWS_PER_TILE)],
                    out_hbm.at[cid, pl.ds(sid * ROWS_PER_TILE, ROWS_PER_TILE)])


_agg_call = functools.partial(
    pl.kernel,
    out_type=jax.ShapeDtypeStruct((2, NPAD, C), jnp.float32),
    mesh=_mesh,
    scratch_types=(
        [pltpu.VMEM((ACH,), jnp.int32)] * 12
        + [pltpu.VMEM((ACH, C), jnp.float32)] * 4
        + [pltpu.VMEM_SHARED((NPAD, C), jnp.float32)]
        + [pltpu.SemaphoreType.DMA] * 8
    ),
)(_agg_body)


_RB = 640


def _g_body(deg_ref, x_ref, w_ref, g_ref):
    p = deg_ref[...]
    s = lax.rsqrt(p[0] + p[1] + 1.0)
    h = jnp.dot(x_ref[...], w_ref[...], preferred_element_type=jnp.float32)
    g_ref[...] = s[:, None] * h


def _g_call(deg_parts, x, W_gcn):
    return pl.pallas_call(
        _g_body,
        grid=(NPAD // _RB,),
        in_specs=[
            pl.BlockSpec((2, _RB), lambda i: (0, i)),
            pl.BlockSpec((_RB, C), lambda i: (i, 0)),
            pl.BlockSpec((C, C), lambda i: (0, 0)),
        ],
        out_specs=pl.BlockSpec((_RB, C), lambda i: (i, 0)),
        out_shape=jax.ShapeDtypeStruct((NPAD, C), jnp.float32),
    )(deg_parts, x, W_gcn)


def _out_body(deg_ref, a_ref, g_ref, b_ref, w_ref, o_ref):
    p = deg_ref[...]
    a = a_ref[...]
    s = lax.rsqrt(p[0] + p[1] + 1.0)
    out1 = s[:, None] * (a[0] + a[1] + g_ref[...]) + b_ref[...]
    o = jnp.dot(out1, w_ref[...], preferred_element_type=jnp.float32)
    o_ref[...] = o.reshape(_RB, HEADS, C)


def _out_call(deg_parts, a_parts, g, b2d, weight):
    return pl.pallas_call(
        _out_body,
        grid=(NPAD // _RB,),
        in_specs=[
            pl.BlockSpec((2, _RB), lambda i: (0, i)),
            pl.BlockSpec((2, _RB, C), lambda i: (0, i, 0)),
            pl.BlockSpec((_RB, C), lambda i: (i, 0)),
            pl.BlockSpec((1, C), lambda i: (0, 0)),
            pl.BlockSpec((C, HEADS * C), lambda i: (0, 0)),
        ],
        out_specs=pl.BlockSpec((_RB, HEADS, C), lambda i: (i, 0, 0)),
        out_shape=jax.ShapeDtypeStruct((N, HEADS, C), jnp.float32),
    )(deg_parts, a_parts, g, b2d, weight)


@jax.jit
def kernel(x, edge_index, old_index, W_gcn, b_gcn, weight):
    row1d = edge_index[0]
    col1d = edge_index[1]
    deg_parts = _deg_call(row1d, col1d)
    g = _g_call(deg_parts, x, W_gcn)
    a_parts = _agg_call(g, row1d, col1d)
    return _out_call(deg_parts, a_parts, g, b_gcn.reshape(1, C), weight)
